# Initial kernel scaffold; baseline (speedup 1.0000x reference)
import jax
import jax.numpy as jnp
from jax.experimental import pallas as pl


def kernel(x, edge_index, batch, params):
    raise NotImplementedError("write your pallas kernel here")


# speedup vs baseline: 4.0174x; 4.0174x over previous
import functools

import jax
import jax.numpy as jnp
from jax import lax
from jax.experimental import pallas as pl
from jax.experimental.pallas import tpu as pltpu
from jax.experimental.pallas import tpu_sc as plsc

_N = 10000
_E = 320000
_H = 128
_OUT = 16
_G = 64
_V = 500
_VPAD = 512

_NC = 2
_NS = 16
_NW = _NC * _NS

_NPAD = 10240
_ROWS_W = _NPAD // _NW
_ROWS_S = _NPAD // _NS
_EPT = _E // _NW
_CH = 79
_EPT_PAD = _CH * 128

_BLK = 640
_NBLK = _NPAD // _BLK

_MESH = plsc.VectorSubcoreMesh(core_axis_name="c", subcore_axis_name="s")


@functools.partial(
    pl.kernel,
    out_type=jax.ShapeDtypeStruct((_NPAD, _H), jnp.float32),
    mesh=_MESH,
    scratch_types=[
        pltpu.VMEM((3, 128), jnp.int32),
        pltpu.VMEM((128, _H), jnp.float32),
        pltpu.SemaphoreType.DMA,
    ],
)
def _sc_embed_gather(tab_hbm, xi_hbm, out_hbm, idxv, rowsv, sem):
    c = lax.axis_index("c")
    s = lax.axis_index("s")
    wid = s * _NC + c
    base = wid * _ROWS_W
    pltpu.sync_copy(xi_hbm.at[wid], idxv)
    for j in range(2):
        pltpu.async_copy(tab_hbm.at[idxv.at[j]], rowsv, sem).wait()
        pltpu.sync_copy(rowsv, out_hbm.at[pl.ds(base + j * 128, 128)])
    pltpu.async_copy(tab_hbm.at[idxv.at[2]], rowsv, sem).wait()
    pltpu.sync_copy(rowsv.at[pl.ds(0, 64)], out_hbm.at[pl.ds(base + 256, 64)])


@functools.partial(
    pl.kernel,
    out_type=jax.ShapeDtypeStruct((_NC, _NPAD, _H), jnp.float32),
    mesh=_MESH,
    scratch_types=[
        pltpu.VMEM_SHARED((_NPAD, _H), jnp.float32),
        pltpu.VMEM((_CH, 128), jnp.int32),
        pltpu.VMEM((_CH, 128), jnp.int32),
        pltpu.VMEM((128, _H), jnp.float32),
        pltpu.SemaphoreType.DMA,
    ],
)
def _sc_edge_agg(h_hbm, zr_hbm, src_hbm, dst_hbm, out_hbm,
                 agg_sh, srcv, dstv, rowsv, sem):
    c = lax.axis_index("c")
    s = lax.axis_index("s")
    wid = s * _NC + c
    pltpu.sync_copy(src_hbm.at[wid], srcv)
    pltpu.sync_copy(dst_hbm.at[wid], dstv)
    pltpu.sync_copy(zr_hbm, agg_sh.at[pl.ds(s * _ROWS_S, _ROWS_S)])
    plsc.subcore_barrier()

    def body(j, carry):
        pltpu.async_copy(h_hbm.at[srcv.at[j]], rowsv, sem).wait()
        pltpu.sync_copy(rowsv, agg_sh.at[dstv.at[j]], add=True)
        return carry

    lax.fori_loop(0, _CH, body, 0)
    plsc.subcore_barrier()
    pltpu.sync_copy(agg_sh.at[pl.ds(s * _ROWS_S, _ROWS_S)],
                    out_hbm.at[c].at[pl.ds(s * _ROWS_S, _ROWS_S)])


def _table_body(emb_ref, w1_ref, b1_ref, w2_ref, b2_ref, out_ref):
    t = jnp.dot(emb_ref[...], w1_ref[...], preferred_element_type=jnp.float32)
    t = jnp.maximum(t + b1_ref[...], 0.0)
    out_ref[...] = (
        jnp.dot(t, w2_ref[...], preferred_element_type=jnp.float32) + b2_ref[...]
    )


def _tc_table(emb_p, w1, b1, w2, b2):
    return pl.pallas_call(
        _table_body,
        out_shape=jax.ShapeDtypeStruct((_VPAD, _H), jnp.float32),
    )(emb_p, w1, b1, w2, b2)


def _mlp_body(h_ref, a0_ref, a1_ref, w1_ref, b1_ref, w2_ref, b2_ref,
              v_ref, stats_ref):
    k = pl.program_id(0)
    t = h_ref[...] + a0_ref[...] + a1_ref[...]
    u = jnp.dot(t, w1_ref[...], preferred_element_type=jnp.float32)
    u = jnp.maximum(u + b1_ref[...], 0.0)
    v = jnp.dot(u, w2_ref[...], preferred_element_type=jnp.float32) + b2_ref[...]
    v_ref[...] = v
    rows = lax.broadcasted_iota(jnp.int32, (_BLK, 1), 0) + k * _BLK
    vm = jnp.where(rows < _N, v, 0.0)
    part = jnp.concatenate(
        [jnp.sum(vm, axis=0, keepdims=True),
         jnp.sum(vm * vm, axis=0, keepdims=True),
         jnp.zeros((6, _H), jnp.float32)], axis=0)

    @pl.when(k == 0)
    def _():
        stats_ref[...] = part

    @pl.when(k > 0)
    def _():
        stats_ref[...] += part


def _tc_mlp(h, a0, a1, w1, b1, w2, b2):
    return pl.pallas_call(
        _mlp_body,
        grid=(_NBLK,),
        in_specs=[
            pl.BlockSpec((_BLK, _H), lambda k: (k, 0)),
            pl.BlockSpec((_BLK, _H), lambda k: (k, 0)),
            pl.BlockSpec((_BLK, _H), lambda k: (k, 0)),
            pl.BlockSpec((_H, _H), lambda k: (0, 0)),
            pl.BlockSpec((1, _H), lambda k: (0, 0)),
            pl.BlockSpec((_H, _H), lambda k: (0, 0)),
            pl.BlockSpec((1, _H), lambda k: (0, 0)),
        ],
        out_specs=[
            pl.BlockSpec((_BLK, _H), lambda k: (k, 0)),
            pl.BlockSpec((8, _H), lambda k: (0, 0)),
        ],
        out_shape=[
            jax.ShapeDtypeStruct((_NPAD, _H), jnp.float32),
            jax.ShapeDtypeStruct((8, _H), jnp.float32),
        ],
        compiler_params=pltpu.CompilerParams(
            dimension_semantics=("arbitrary",)),
    )(h, a0, a1, w1, b1, w2, b2)


def _norm_body(v_ref, stats_ref, g_ref, beta_ref, out_ref):
    s = stats_ref[...]
    mu = s[0:1, :] / float(_N)
    var = s[1:2, :] / float(_N) - mu * mu
    inv = lax.rsqrt(var + 1e-5)
    out_ref[...] = jnp.maximum(
        (v_ref[...] - mu) * inv * g_ref[...] + beta_ref[...], 0.0)


def _tc_norm(v, stats, g, beta):
    return pl.pallas_call(
        _norm_body,
        grid=(_NBLK,),
        in_specs=[
            pl.BlockSpec((_BLK, _H), lambda k: (k, 0)),
            pl.BlockSpec((8, _H), lambda k: (0, 0)),
            pl.BlockSpec((1, _H), lambda k: (0, 0)),
            pl.BlockSpec((1, _H), lambda k: (0, 0)),
        ],
        out_specs=pl.BlockSpec((_BLK, _H), lambda k: (k, 0)),
        out_shape=jax.ShapeDtypeStruct((_NPAD, _H), jnp.float32),
        compiler_params=pltpu.CompilerParams(
            dimension_semantics=("arbitrary",)),
    )(v, stats, g, beta)


def _pool_body(h_ref, b_ref, out_ref):
    k = pl.program_id(0)
    b = b_ref[0, 0, :]
    gids = lax.broadcasted_iota(jnp.int32, (_G, _BLK), 0)
    oh = (gids == b[None, :]).astype(jnp.float32)
    part = jnp.dot(oh, h_ref[...], preferred_element_type=jnp.float32)

    @pl.when(k == 0)
    def _():
        out_ref[...] = part

    @pl.when(k > 0)
    def _():
        out_ref[...] += part


def _tc_pool(h, batch3):
    return pl.pallas_call(
        _pool_body,
        grid=(_NBLK,),
        in_specs=[
            pl.BlockSpec((_BLK, _H), lambda k: (k, 0)),
            pl.BlockSpec((1, 1, _BLK), lambda k: (k, 0, 0)),
        ],
        out_specs=pl.BlockSpec((_G, _H), lambda k: (0, 0)),
        out_shape=jax.ShapeDtypeStruct((_G, _H), jnp.float32),
        compiler_params=pltpu.CompilerParams(
            dimension_semantics=("arbitrary",)),
    )(h, batch3)


def _final_body(p_ref, w1_ref, b1_ref, w2_ref, b2_ref, out_ref):
    r = jnp.dot(p_ref[...], w1_ref[...], preferred_element_type=jnp.float32)
    r = jnp.maximum(r + b1_ref[...], 0.0)
    out_ref[...] = (
        jnp.dot(r, w2_ref[...], preferred_element_type=jnp.float32) + b2_ref[...]
    )


def _tc_final(pooled, w1, b1, w2, b2):
    return pl.pallas_call(
        _final_body,
        out_shape=jax.ShapeDtypeStruct((_G, _OUT), jnp.float32),
    )(pooled, w1, b1, w2, b2)


def _row(b):
    return b.reshape(1, -1)


def kernel(x, edge_index, batch, params):
    p = params
    src, dst = edge_index[0], edge_index[1]

    emb_p = jnp.pad(p['emb'], ((0, _VPAD - _V), (0, 0)))
    xi = jnp.pad(
        jnp.pad(x, (0, _NPAD - _N)).reshape(_NW, _ROWS_W),
        ((0, 0), (0, 384 - _ROWS_W)),
    ).reshape(_NW, 3, 128)
    src3 = jnp.pad(
        src.reshape(_NW, _EPT), ((0, 0), (0, _EPT_PAD - _EPT)),
    ).reshape(_NW, _CH, 128)
    dst3 = jnp.pad(
        dst.reshape(_NW, _EPT), ((0, 0), (0, _EPT_PAD - _EPT)),
        constant_values=_N,
    ).reshape(_NW, _CH, 128)
    batch3 = jnp.pad(batch, (0, _NPAD - _N), constant_values=_G).reshape(
        _NS, 1, _BLK)
    zr = jnp.zeros((_ROWS_S, _H), jnp.float32)

    tab = _tc_table(emb_p, p['Wi1'], _row(p['bi1']), p['Wi2'], _row(p['bi2']))
    h = _sc_embed_gather(tab, xi)
    for cp in p['convs']:
        agg = _sc_edge_agg(h, zr, src3, dst3)
        v, stats = _tc_mlp(h, agg[0], agg[1], cp['W1'], _row(cp['b1']),
                           cp['W2'], _row(cp['b2']))
        h = _tc_norm(v, stats, _row(cp['g']), _row(cp['beta']))
    pooled = _tc_pool(h, batch3)
    return _tc_final(pooled, p['Wf1'], _row(p['bf1']),
                     p['Wf2'], _row(p['bf2']))
